# Initial kernel scaffold; baseline (speedup 1.0000x reference)
import jax
import jax.numpy as jnp
from jax.experimental import pallas as pl


def kernel(s, v, pos, edge_index, W1, b1, W2, b2, Ws, bs, Wv, bv):
    raise NotImplementedError("write your pallas kernel here")


# speedup vs baseline: 33.4689x; 33.4689x over previous
import jax
import jax.numpy as jnp
from jax import lax
from jax.experimental import pallas as pl
from jax.experimental.pallas import tpu as pltpu
from jax.experimental.pallas import tpu_sc as plsc

N = 10000
E = 320000
HIDDEN = 128
NRBF = 32
CUTOFF = 5.0

NC = 2
NS = 16
NW = NC * NS

TASK = 128
CH = 2 * TASK
ROWS_PER_TILE = N // NS

CHUNK_TASKS = (640, 640, 640, 580)
BE = 2560


def _make_gather_body(nt):
    tpw = nt // NW
    nch = tpw // 2

    def body(s_hbm, pos_hbm, src_hbm, dst_hbm,
             si_hbm, sj_hbm, pi_hbm, pj_hbm,
             ixs, ixd, s_s, s_d, p_s, p_d, semA, semB, semW):
        wid = lax.axis_index("s") * NC + lax.axis_index("c")
        start = wid * tpw
        pltpu.sync_copy(src_hbm.at[pl.ds(start, tpw)], ixs)
        pltpu.sync_copy(dst_hbm.at[pl.ds(start, tpw)], ixd)

        @pl.loop(0, nch)
        def _chunk(j):
            k = 2 * j
            ebase = (start + k) * TASK
            for (ix, sbuf, pbuf, sem) in ((ixs, s_s, p_s, semA),
                                          (ixd, s_d, p_d, semB)):
                pltpu.async_copy(s_hbm.at[ix.at[k]], sbuf.at[pl.ds(0, TASK)],
                                 sem)
                pltpu.async_copy(s_hbm.at[ix.at[k + 1]],
                                 sbuf.at[pl.ds(TASK, TASK)], sem)
                pltpu.async_copy(pos_hbm.at[ix.at[k]],
                                 pbuf.at[pl.ds(0, TASK)], sem)
                pltpu.async_copy(pos_hbm.at[ix.at[k + 1]],
                                 pbuf.at[pl.ds(TASK, TASK)], sem)
            for (ix, sbuf, pbuf, sem, s_out, p_out) in (
                    (ixs, s_s, p_s, semA, si_hbm, pi_hbm),
                    (ixd, s_d, p_d, semB, sj_hbm, pj_hbm)):
                pltpu.make_async_copy(s_hbm.at[ix.at[k]],
                                      sbuf.at[pl.ds(0, TASK)], sem).wait()
                pltpu.make_async_copy(s_hbm.at[ix.at[k + 1]],
                                      sbuf.at[pl.ds(TASK, TASK)], sem).wait()
                pltpu.make_async_copy(pos_hbm.at[ix.at[k]],
                                      pbuf.at[pl.ds(0, TASK)], sem).wait()
                pltpu.make_async_copy(pos_hbm.at[ix.at[k + 1]],
                                      pbuf.at[pl.ds(TASK, TASK)], sem).wait()
                pltpu.async_copy(sbuf, s_out.at[pl.ds(ebase, CH)], semW)
                pltpu.async_copy(pbuf, p_out.at[pl.ds(ebase, CH)], semW)
            for (sbuf, pbuf, s_out, p_out) in ((s_s, p_s, si_hbm, pi_hbm),
                                               (s_d, p_d, sj_hbm, pj_hbm)):
                pltpu.make_async_copy(sbuf, s_out.at[pl.ds(ebase, CH)],
                                      semW).wait()
                pltpu.make_async_copy(pbuf, p_out.at[pl.ds(ebase, CH)],
                                      semW).wait()

        nleft = nt - NW * tpw
        if nleft:
            @pl.when(wid < nleft)
            def _leftover():
                t = NW * tpw + wid
                ebase = t * TASK
                pltpu.sync_copy(src_hbm.at[t], ixs.at[0])
                pltpu.sync_copy(dst_hbm.at[t], ixd.at[0])
                pltpu.async_copy(s_hbm.at[ixs.at[0]], s_s.at[pl.ds(0, TASK)],
                                 semA)
                pltpu.async_copy(pos_hbm.at[ixs.at[0]],
                                 p_s.at[pl.ds(0, TASK)], semA)
                pltpu.async_copy(s_hbm.at[ixd.at[0]], s_d.at[pl.ds(0, TASK)],
                                 semB)
                pltpu.async_copy(pos_hbm.at[ixd.at[0]],
                                 p_d.at[pl.ds(0, TASK)], semB)
                pltpu.make_async_copy(s_hbm.at[ixs.at[0]],
                                      s_s.at[pl.ds(0, TASK)], semA).wait()
                pltpu.make_async_copy(pos_hbm.at[ixs.at[0]],
                                      p_s.at[pl.ds(0, TASK)], semA).wait()
                pltpu.make_async_copy(s_hbm.at[ixd.at[0]],
                                      s_d.at[pl.ds(0, TASK)], semB).wait()
                pltpu.make_async_copy(pos_hbm.at[ixd.at[0]],
                                      p_d.at[pl.ds(0, TASK)], semB).wait()
                pltpu.sync_copy(s_s.at[pl.ds(0, TASK)],
                                si_hbm.at[pl.ds(ebase, TASK)])
                pltpu.sync_copy(p_s.at[pl.ds(0, TASK)],
                                pi_hbm.at[pl.ds(ebase, TASK)])
                pltpu.sync_copy(s_d.at[pl.ds(0, TASK)],
                                sj_hbm.at[pl.ds(ebase, TASK)])
                pltpu.sync_copy(p_d.at[pl.ds(0, TASK)],
                                pj_hbm.at[pl.ds(ebase, TASK)])

    return body, tpw


def _sc_gather(s, pos_pad, src2d, dst2d):
    nt = src2d.shape[0]
    ne = nt * TASK
    body, tpw = _make_gather_body(nt)
    mesh = plsc.VectorSubcoreMesh(core_axis_name="c", subcore_axis_name="s")
    f32 = jnp.float32
    out_type = (
        jax.ShapeDtypeStruct((ne, HIDDEN), f32),
        jax.ShapeDtypeStruct((ne, HIDDEN), f32),
        jax.ShapeDtypeStruct((ne, 16), f32),
        jax.ShapeDtypeStruct((ne, 16), f32),
    )
    scratch = [
        pltpu.VMEM((tpw, TASK), jnp.int32),
        pltpu.VMEM((tpw, TASK), jnp.int32),
        pltpu.VMEM((CH, HIDDEN), f32),
        pltpu.VMEM((CH, HIDDEN), f32),
        pltpu.VMEM((CH, 16), f32),
        pltpu.VMEM((CH, 16), f32),
        pltpu.SemaphoreType.DMA,
        pltpu.SemaphoreType.DMA,
        pltpu.SemaphoreType.DMA,
    ]
    return pl.kernel(
        body, out_type=out_type, mesh=mesh, scratch_types=scratch,
        compiler_params=pltpu.CompilerParams(use_tc_tiling_on_sc=False),
        name=f"painn_sc_gather_{nt}",
    )(s, pos_pad, src2d, dst2d)


def _make_scatter_body(nt):
    tpt = nt // NS
    sch = tpt // 2
    npair = sch // 2
    nleft = nt - NS * tpt
    HH = HIDDEN // 2

    def body(dst_hbm,
             v0_hbm, v1_hbm, v2_hbm, v3_hbm,
             i0_hbm, i1_hbm, i2_hbm, i3_hbm,
             o0_hbm, o1_hbm, o2_hbm, o3_hbm,
             acc, idx_b, idx1, valA, valB, semA, semB):
        c = lax.axis_index("c")
        sid = lax.axis_index("s")
        vals = (v0_hbm, v1_hbm, v2_hbm, v3_hbm)
        inits = (i0_hbm, i1_hbm, i2_hbm, i3_hbm)
        outs = (o0_hbm, o1_hbm, o2_hbm, o3_hbm)
        row0 = sid * ROWS_PER_TILE
        start = sid * tpt
        pltpu.sync_copy(dst_hbm.at[pl.ds(start, tpt)], idx_b)

        for r in range(4):
            for ci in range(2):
                q = 2 * r + ci
                a, h = q // 2, q % 2

                @pl.when(c == ci)
                def _init(a=a, h=h):
                    pltpu.sync_copy(
                        inits[a].at[pl.ds(row0, ROWS_PER_TILE),
                                    pl.ds(h * HH, HH)],
                        acc.at[pl.ds(row0, ROWS_PER_TILE)])

            plsc.subcore_barrier()
            for ci in range(2):
                q = 2 * r + ci
                a, h = q // 2, q % 2

                @pl.when(c == ci)
                def _scatter(a=a, h=h):
                    v_hbm = vals[a]

                    def _src(ch):
                        return v_hbm.at[pl.ds((start + 2 * ch) * TASK, CH),
                                        pl.ds(h * HH, HH)]

                    pltpu.async_copy(_src(0), valA, semA)

                    @pl.loop(0, npair)
                    def _pair(m):
                        chA = 2 * m
                        chB = 2 * m + 1
                        pltpu.async_copy(_src(chB), valB, semB)
                        pltpu.make_async_copy(_src(chA), valA, semA).wait()
                        pltpu.sync_copy(valA.at[pl.ds(0, TASK)],
                                        acc.at[idx_b.at[2 * chA]], add=True)
                        pltpu.sync_copy(valA.at[pl.ds(TASK, TASK)],
                                        acc.at[idx_b.at[2 * chA + 1]],
                                        add=True)

                        @pl.when(m < npair - 1)
                        def _fire_next():
                            pltpu.async_copy(_src(chA + 2), valA, semA)

                        pltpu.make_async_copy(_src(chB), valB, semB).wait()
                        pltpu.sync_copy(valB.at[pl.ds(0, TASK)],
                                        acc.at[idx_b.at[2 * chB]], add=True)
                        pltpu.sync_copy(valB.at[pl.ds(TASK, TASK)],
                                        acc.at[idx_b.at[2 * chB + 1]],
                                        add=True)

                    if nleft:
                        @pl.when(sid < nleft)
                        def _leftover():
                            t = NS * tpt + sid
                            pltpu.sync_copy(dst_hbm.at[t], idx1)
                            pltpu.sync_copy(
                                v_hbm.at[pl.ds(t * TASK, TASK),
                                         pl.ds(h * HH, HH)],
                                valA.at[pl.ds(0, TASK)])
                            pltpu.sync_copy(valA.at[pl.ds(0, TASK)],
                                            acc.at[idx1], add=True)

            plsc.subcore_barrier()
            for ci in range(2):
                q = 2 * r + ci
                a, h = q // 2, q % 2

                @pl.when(c == ci)
                def _writeback(a=a, h=h):
                    pltpu.sync_copy(
                        acc.at[pl.ds(row0, ROWS_PER_TILE)],
                        outs[a].at[pl.ds(row0, ROWS_PER_TILE),
                                   pl.ds(h * HH, HH)])

            if r < 3:
                plsc.subcore_barrier()

    return body, tpt


def _sc_scatter(dst2d, ds_e, dvx_e, dvy_e, dvz_e, s, vx, vy, vz):
    nt = dst2d.shape[0]
    body, tpt = _make_scatter_body(nt)
    mesh = plsc.VectorSubcoreMesh(core_axis_name="c", subcore_axis_name="s")
    f32 = jnp.float32
    out_type = tuple(jax.ShapeDtypeStruct((N, HIDDEN), f32) for _ in range(4))
    scratch = [
        pltpu.VMEM_SHARED((N, HIDDEN // 2), f32),
        pltpu.VMEM((tpt, TASK), jnp.int32),
        pltpu.VMEM((TASK,), jnp.int32),
        pltpu.VMEM((CH, HIDDEN // 2), f32),
        pltpu.VMEM((CH, HIDDEN // 2), f32),
        pltpu.SemaphoreType.DMA,
        pltpu.SemaphoreType.DMA,
    ]
    return pl.kernel(
        body, out_type=out_type, mesh=mesh, scratch_types=scratch,
        compiler_params=pltpu.CompilerParams(use_tc_tiling_on_sc=False),
        name=f"painn_sc_scatter_{nt}",
    )(dst2d, ds_e, dvx_e, dvy_e, dvz_e, s, vx, vy, vz)


def _mlp_body(si_ref, sj_ref, pi_ref, pj_ref,
              w1a_ref, w1b_ref, w1c_ref, b1_ref, w2_ref, b2_ref,
              ws_ref, bs_ref, wv_ref, bv_ref,
              ds_ref, dvx_ref, dvy_ref, dvz_ref):
    f32 = jnp.float32
    rij = pj_ref[...] - pi_ref[...]
    d2 = jnp.sum(rij * rij, axis=1, keepdims=True)
    dist = jnp.sqrt(d2)
    inv = jnp.where(dist > 0.0, 1.0 / jnp.where(dist > 0.0, dist, 1.0), 0.0)
    dirs = rij * inv

    delta = float(CUTOFF / (NRBF - 1))
    centers = lax.broadcasted_iota(jnp.int32, (1, NRBF), 1).astype(f32) * delta
    scale = 1.0 / (delta + 1e-08)
    diff = (dist - centers) * scale
    rbf = jnp.exp(-(diff * diff))

    acc = jax.lax.dot_general(si_ref[...], w1a_ref[...],
                              (((1,), (0,)), ((), ())),
                              preferred_element_type=f32)
    acc += jax.lax.dot_general(sj_ref[...], w1b_ref[...],
                               (((1,), (0,)), ((), ())),
                               preferred_element_type=f32)
    acc += jax.lax.dot_general(rbf, w1c_ref[...],
                               (((1,), (0,)), ((), ())),
                               preferred_element_type=f32)
    acc += b1_ref[...]
    h = acc * jax.nn.sigmoid(acc)

    acc2 = jax.lax.dot_general(h, w2_ref[...], (((1,), (0,)), ((), ())),
                               preferred_element_type=f32) + b2_ref[...]
    h2 = acc2 * jax.nn.sigmoid(acc2)

    ds = jax.lax.dot_general(h2, ws_ref[...], (((1,), (0,)), ((), ())),
                             preferred_element_type=f32) + bs_ref[...]
    dvm = jax.lax.dot_general(h2, wv_ref[...], (((1,), (0,)), ((), ())),
                              preferred_element_type=f32) + bv_ref[...]

    ds_ref[...] = ds
    dvx_ref[...] = dirs[:, 0:1] * dvm
    dvy_ref[...] = dirs[:, 1:2] * dvm
    dvz_ref[...] = dirs[:, 2:3] * dvm


def _tc_mlp(si, sj, pi, pj, W1a, W1b, W1c, b1, W2, b2, Ws, bs, Wv, bv):
    f32 = jnp.float32
    ne = si.shape[0]
    n_blocks = ne // BE
    edge_spec = lambda w: pl.BlockSpec((BE, w), lambda i: (i, 0))
    full = lambda a: pl.BlockSpec(a.shape, lambda i: tuple(0 for _ in a.shape))
    grid_spec = pl.GridSpec(
        grid=(n_blocks,),
        in_specs=[
            edge_spec(HIDDEN), edge_spec(HIDDEN), edge_spec(16), edge_spec(16),
            full(W1a), full(W1b), full(W1c), full(b1), full(W2), full(b2),
            full(Ws), full(bs), full(Wv), full(bv),
        ],
        out_specs=[edge_spec(HIDDEN)] * 4,
    )
    out_shape = tuple(jax.ShapeDtypeStruct((ne, HIDDEN), f32)
                      for _ in range(4))
    return pl.pallas_call(
        _mlp_body, grid_spec=grid_spec, out_shape=out_shape,
        name=f"painn_tc_mlp_{ne}",
    )(si, sj, pi, pj, W1a, W1b, W1c, b1, W2, b2, Ws, bs, Wv, bv)


@jax.jit
def kernel(s, v, pos, edge_index, W1, b1, W2, b2, Ws, bs, Wv, bv):
    f32 = jnp.float32
    pos_pad = jnp.pad(pos.astype(f32), ((0, 0), (0, 13)))
    W1a, W1b, W1c = W1[:HIDDEN], W1[HIDDEN:2 * HIDDEN], W1[2 * HIDDEN:]
    b1r, b2r = b1.reshape(1, -1), b2.reshape(1, -1)
    bsr, bvr = bs.reshape(1, -1), bv.reshape(1, -1)
    vx, vy, vz = v[:, 0, :], v[:, 1, :], v[:, 2, :]

    srcs, dsts = [], []
    off = 0
    for nt in CHUNK_TASKS:
        ne = nt * TASK
        srcs.append(edge_index[0, off:off + ne].reshape(nt, TASK))
        dsts.append(edge_index[1, off:off + ne].reshape(nt, TASK))
        off += ne

    gs = [_sc_gather(s, pos_pad, sc, dc) for sc, dc in zip(srcs, dsts)]
    ms = [_tc_mlp(*g, W1a, W1b, W1c, b1r, W2, b2r, Ws, bsr, Wv, bvr)
          for g in gs]
    acc = (s, vx, vy, vz)
    for dc, m in zip(dsts, ms):
        acc = _sc_scatter(dc, *m, *acc)

    s_out, vxo, vyo, vzo = acc
    v_out = jnp.stack([vxo, vyo, vzo], axis=1)
    return (s_out, v_out)
